# baseline (speedup 1.0000x reference)
import functools

import jax
import jax.numpy as jnp
from jax import lax
from jax.experimental import pallas as pl
from jax.experimental.pallas import tpu as pltpu
from jax.experimental.pallas import tpu_sc as plsc

D_MODEL = 64
SCALE = 8.0
NUM_WORKERS = 32
CHUNK = 128
NBUF = 4


@functools.lru_cache(maxsize=None)
def _build_gather(seq_len: int):
    n_chunks = seq_len
    assert n_chunks % NBUF == 0

    mesh = plsc.VectorSubcoreMesh(core_axis_name="c", subcore_axis_name="s")

    scratch = [pltpu.VMEM((n_chunks, CHUNK), jnp.int32)]
    scratch += [pltpu.VMEM((CHUNK, D_MODEL), jnp.float32) for _ in range(NBUF)]
    scratch += [pltpu.SemaphoreType.DMA for _ in range(2 * NBUF)]

    @functools.partial(
        pl.kernel,
        mesh=mesh,
        out_type=jax.ShapeDtypeStruct(
            (seq_len, NUM_WORKERS, CHUNK, D_MODEL), jnp.float32
        ),
        scratch_types=scratch,
        compiler_params=pltpu.CompilerParams(
            needs_layout_passes=False, use_tc_tiling_on_sc=False
        ),
    )
    def gather_kernel(xt_hbm, tab_hbm, out_hbm, idx_v, *rest):
        gbufs = rest[:NBUF]
        gsems = rest[NBUF : 2 * NBUF]
        osems = rest[2 * NBUF :]
        wid = lax.axis_index("s") * 2 + lax.axis_index("c")

        pltpu.sync_copy(xt_hbm.at[wid], idx_v)

        for b in range(NBUF):
            pltpu.async_copy(tab_hbm.at[idx_v.at[b]], gbufs[b], gsems[b])

        def process(t, b):
            pltpu.make_async_copy(
                tab_hbm.at[idx_v.at[t]], gbufs[b], gsems[b]
            ).wait()
            pltpu.async_copy(gbufs[b], out_hbm.at[t, wid], osems[b])
            nxt = t + NBUF

            @pl.when(nxt < n_chunks)
            def _(b=b, t=t, nxt=nxt):
                pltpu.make_async_copy(
                    gbufs[b], out_hbm.at[t, wid], osems[b]
                ).wait()
                pltpu.async_copy(
                    tab_hbm.at[idx_v.at[nxt]], gbufs[b], gsems[b]
                )

        def outer(g, carry):
            for b in range(NBUF):
                process(g * NBUF + b, b)
            return carry

        lax.fori_loop(0, n_chunks // NBUF, outer, 0)

        for b in range(NBUF):
            pltpu.make_async_copy(
                gbufs[b], out_hbm.at[0, wid], osems[b]
            ).wait()

    return gather_kernel


@functools.lru_cache(maxsize=None)
def _build_xform(seq_len: int):
    def body(x_ref, o_ref):
        ii = lax.broadcasted_iota(jnp.int32, (CHUNK, CHUNK), 0)
        jj = lax.broadcasted_iota(jnp.int32, (CHUNK, CHUNK), 1)
        ident = jnp.where(ii == jj, SCALE, 0.0).astype(jnp.float32)
        for w in range(NUM_WORKERS):
            tr = lax.dot_general(
                x_ref[0, w],
                ident,
                dimension_numbers=(((0,), (0,)), ((), ())),
                preferred_element_type=jnp.float32,
                precision=lax.Precision.HIGHEST,
            )
            for g in range(8):
                o_ref[0, g, w] = tr[g * 8 : (g + 1) * 8]

    return pl.pallas_call(
        body,
        grid=(seq_len,),
        in_specs=[
            pl.BlockSpec(
                (1, NUM_WORKERS, CHUNK, D_MODEL), lambda t: (t, 0, 0, 0)
            )
        ],
        out_specs=pl.BlockSpec(
            (1, 8, NUM_WORKERS, 8, CHUNK), lambda t: (t, 0, 0, 0, 0)
        ),
        out_shape=jax.ShapeDtypeStruct(
            (seq_len, 8, NUM_WORKERS, 8, CHUNK), jnp.float32
        ),
    )


def kernel(x, table):
    b, t = x.shape
    vocab, d = table.shape
    xt = x.T.astype(jnp.int32).reshape(t, NUM_WORKERS, CHUNK).transpose(1, 0, 2)
    rows = _build_gather(t)(xt, table)
    out5 = _build_xform(t)(rows)
    return out5.transpose(2, 4, 0, 1, 3).reshape(b, t, d)


# speedup vs baseline: 1.1794x; 1.1794x over previous
import functools

import jax
import jax.numpy as jnp
from jax import lax
from jax.experimental import pallas as pl
from jax.experimental.pallas import tpu as pltpu
from jax.experimental.pallas import tpu_sc as plsc

D_MODEL = 64
SCALE = 8.0
NUM_WORKERS = 32
CHUNK = 128
NBUF = 4


@functools.lru_cache(maxsize=None)
def _build(n_rows: int):
    rows_per_w = n_rows // NUM_WORKERS
    n_chunks = rows_per_w // CHUNK
    assert n_chunks % NBUF == 0

    mesh = plsc.VectorSubcoreMesh(core_axis_name="c", subcore_axis_name="s")

    scratch = [pltpu.VMEM((n_chunks, CHUNK), jnp.int32)]
    scratch += [pltpu.VMEM((CHUNK, D_MODEL), jnp.float32) for _ in range(NBUF)]
    scratch += [pltpu.SemaphoreType.DMA for _ in range(2 * NBUF)]

    @functools.partial(
        pl.kernel,
        mesh=mesh,
        out_type=jax.ShapeDtypeStruct(
            (NUM_WORKERS, n_chunks, CHUNK, D_MODEL), jnp.float32
        ),
        scratch_types=scratch,
        compiler_params=pltpu.CompilerParams(use_tc_tiling_on_sc=False),
    )
    def emb_kernel(x_hbm, tab_hbm, out_hbm, idx_v, *rest):
        bufs = rest[:NBUF]
        gsems = rest[NBUF : 2 * NBUF]
        osems = rest[2 * NBUF :]
        wid = lax.axis_index("s") * 2 + lax.axis_index("c")

        pltpu.sync_copy(x_hbm.at[wid], idx_v)

        for b in range(NBUF):
            pltpu.async_copy(tab_hbm.at[idx_v.at[b]], bufs[b], gsems[b])

        def outer(g, carry):
            for b in range(NBUF):
                j = g * NBUF + b
                pltpu.make_async_copy(
                    tab_hbm.at[idx_v.at[j]], bufs[b], gsems[b]
                ).wait()

                def scale(r, c, buf=bufs[b]):
                    for rr in range(4):
                        for u in range(4):
                            sl = (r * 4 + rr, pl.ds(u * 16, 16))
                            buf[sl] = buf[sl] * SCALE
                    return c

                lax.fori_loop(0, CHUNK // 4, scale, 0)

                pltpu.async_copy(bufs[b], out_hbm.at[wid, j], osems[b])
                nxt = j + NBUF

                @pl.when(nxt < n_chunks)
                def _(b=b, j=j, nxt=nxt):
                    pltpu.make_async_copy(
                        bufs[b], out_hbm.at[wid, j], osems[b]
                    ).wait()
                    pltpu.async_copy(
                        tab_hbm.at[idx_v.at[nxt]], bufs[b], gsems[b]
                    )

            return carry

        lax.fori_loop(0, n_chunks // NBUF, outer, 0)

        for b in range(NBUF):
            pltpu.make_async_copy(bufs[b], out_hbm.at[wid, 0], osems[b]).wait()

    return emb_kernel


def kernel(x, table):
    b, t = x.shape
    n_rows = b * t
    xr = x.reshape(NUM_WORKERS, n_rows // (NUM_WORKERS * CHUNK), CHUNK).astype(
        jnp.int32
    )
    out = _build(n_rows)(xr, table)
    return out.reshape(b, t, D_MODEL)
